# initial kernel scaffold (unmeasured)
import jax
import jax.numpy as jnp
from jax import lax
from jax.experimental import pallas as pl
from jax.experimental.pallas import tpu as pltpu

N_DEV = 4


def kernel(x, w_mat):
    m_per, k = x.shape
    _, n_per = w_mat.shape
    m_total = N_DEV * m_per

    def body(x_ref, w_ref, out_ref,
             comm_ref, send_sems, recv_sems,
             amax_src, amax_dst, amax_send_sems, amax_recv_sems):
        my = lax.axis_index("i")
        left = lax.rem(my + N_DEV - 1, N_DEV)
        right = lax.rem(my + 1, N_DEV)

        barrier_sem = pltpu.get_barrier_semaphore()
        for nbr in (left, right):
            pl.semaphore_signal(barrier_sem, inc=1, device_id=(nbr,),
                                device_id_type=pl.DeviceIdType.MESH)
        pl.semaphore_wait(barrier_sem, 2)

        w_bf = w_ref[...].astype(jnp.bfloat16)
        comm_ref[0] = x_ref[...].astype(jnp.bfloat16)

        def block_out(xb):
            y = lax.dot_general(xb, w_bf, (((1,), (0,)), ((), ())),
                                preferred_element_type=jnp.float32)
            return jnp.maximum(y, 0.0)

        amax = jnp.float32(0.0)
        for h in range(N_DEV - 1):
            s, r = h % 2, (h + 1) % 2
            rdma = pltpu.make_async_remote_copy(
                src_ref=comm_ref.at[s],
                dst_ref=comm_ref.at[r],
                send_sem=send_sems.at[s],
                recv_sem=recv_sems.at[r],
                device_id=(right,),
                device_id_type=pl.DeviceIdType.MESH,
            )
            rdma.start()
            origin = lax.rem(my + N_DEV - h, N_DEV)
            yb = block_out(comm_ref[s])
            out_ref[pl.ds(origin * m_per, m_per), :] = yb
            amax = jnp.maximum(amax, jnp.max(yb))
            rdma.wait()
        s_last = (N_DEV - 1) % 2
        origin = lax.rem(my + 1, N_DEV)
        yb = block_out(comm_ref[s_last])
        out_ref[pl.ds(origin * m_per, m_per), :] = yb
        amax = jnp.maximum(amax, jnp.max(yb))

        amax_src[...] = jnp.full((8, 128), amax, jnp.float32)
        sends = []
        for off in range(1, N_DEV):
            dst = lax.rem(my + off, N_DEV)
            snd = pltpu.make_async_remote_copy(
                src_ref=amax_src,
                dst_ref=amax_dst.at[my],
                send_sem=amax_send_sems.at[off - 1],
                recv_sem=amax_recv_sems.at[my],
                device_id=(dst,),
                device_id_type=pl.DeviceIdType.MESH,
            )
            snd.start()
            sends.append(snd)
        gmax = amax
        for off in range(1, N_DEV):
            src = lax.rem(my + off, N_DEV)
            rcv = pltpu.make_async_remote_copy(
                src_ref=amax_src,
                dst_ref=amax_dst.at[src],
                send_sem=amax_send_sems.at[off - 1],
                recv_sem=amax_recv_sems.at[src],
                device_id=(src,),
                device_id_type=pl.DeviceIdType.MESH,
            )
            rcv.wait_recv()
            gmax = jnp.maximum(gmax, jnp.max(amax_dst[src]))
        for snd in sends:
            snd.wait_send()

        scale = gmax / 127.0
        q = jnp.clip(jnp.round(out_ref[...] / scale), -127.0, 127.0)
        out_ref[...] = q * scale

    return pl.pallas_call(
        body,
        out_shape=jax.ShapeDtypeStruct((m_total, n_per), jnp.float32),
        in_specs=[
            pl.BlockSpec(memory_space=pltpu.VMEM),
            pl.BlockSpec(memory_space=pltpu.VMEM),
        ],
        out_specs=pl.BlockSpec(memory_space=pltpu.VMEM),
        scratch_shapes=[
            pltpu.VMEM((2, m_per, k), jnp.bfloat16),
            pltpu.SemaphoreType.DMA((2,)),
            pltpu.SemaphoreType.DMA((2,)),
            pltpu.VMEM((8, 128), jnp.float32),
            pltpu.VMEM((N_DEV, 8, 128), jnp.float32),
            pltpu.SemaphoreType.DMA((N_DEV - 1,)),
            pltpu.SemaphoreType.DMA((N_DEV,)),
        ],
        compiler_params=pltpu.CompilerParams(collective_id=0),
    )(x, w_mat)


# baseline (device time: 305920 ns/iter reference)
import jax
import jax.numpy as jnp
from jax import lax
from jax.experimental import pallas as pl
from jax.experimental.pallas import tpu as pltpu

N_DEV = 4


def kernel(x, w_mat):
    m_per, k = x.shape
    _, n_per = w_mat.shape
    m_total = N_DEV * m_per

    def body(x_ref, w_ref, out_ref,
             comm_ref, send_sems, recv_sems,
             amax_src, amax_dst, amax_send_sems, amax_recv_sems):
        my = lax.axis_index("i")
        left = lax.rem(my + N_DEV - 1, N_DEV)
        right = lax.rem(my + 1, N_DEV)

        barrier_sem = pltpu.get_barrier_semaphore()
        for nbr in (left, right):
            pl.semaphore_signal(barrier_sem, inc=1, device_id=(nbr,),
                                device_id_type=pl.DeviceIdType.MESH)
        pl.semaphore_wait(barrier_sem, 2)

        w_bf = w_ref[...].astype(jnp.bfloat16)
        comm_ref[0] = x_ref[...].astype(jnp.bfloat16)

        def block_out(xb):
            y = lax.dot_general(xb, w_bf, (((1,), (0,)), ((), ())),
                                preferred_element_type=jnp.float32)
            return jnp.maximum(y, 0.0)

        amax = jnp.float32(0.0)
        for h in range(N_DEV - 1):
            s, r = h % 2, (h + 1) % 2
            rdma = pltpu.make_async_remote_copy(
                src_ref=comm_ref.at[s],
                dst_ref=comm_ref.at[r],
                send_sem=send_sems.at[s],
                recv_sem=recv_sems.at[r],
                device_id=(right,),
                device_id_type=pl.DeviceIdType.MESH,
            )
            rdma.start()
            origin = lax.rem(my + N_DEV - h, N_DEV)
            yb = block_out(comm_ref[s])
            out_ref[pl.ds(origin * m_per, m_per), :] = yb
            amax = jnp.maximum(amax, jnp.max(yb))
            rdma.wait()
        s_last = (N_DEV - 1) % 2
        origin = lax.rem(my + 1, N_DEV)
        yb = block_out(comm_ref[s_last])
        out_ref[pl.ds(origin * m_per, m_per), :] = yb
        amax = jnp.maximum(amax, jnp.max(yb))

        amax_src[...] = jnp.full((8, 128), amax, jnp.float32)
        sends = []
        for off in range(1, N_DEV):
            dst = lax.rem(my + off, N_DEV)
            snd = pltpu.make_async_remote_copy(
                src_ref=amax_src,
                dst_ref=amax_dst.at[my],
                send_sem=amax_send_sems.at[off - 1],
                recv_sem=amax_recv_sems.at[my],
                device_id=(dst,),
                device_id_type=pl.DeviceIdType.MESH,
            )
            snd.start()
            sends.append(snd)
        gmax = amax
        for off in range(1, N_DEV):
            src = lax.rem(my + off, N_DEV)
            rcv = pltpu.make_async_remote_copy(
                src_ref=amax_src,
                dst_ref=amax_dst.at[src],
                send_sem=amax_send_sems.at[off - 1],
                recv_sem=amax_recv_sems.at[src],
                device_id=(src,),
                device_id_type=pl.DeviceIdType.MESH,
            )
            rcv.wait_recv()
            gmax = jnp.maximum(gmax, jnp.max(amax_dst[src]))
        for snd in sends:
            snd.wait_send()

        scale = gmax / 127.0
        q = jnp.clip(jnp.round(out_ref[...] / scale), -127.0, 127.0)
        out_ref[...] = q * scale

    return pl.pallas_call(
        body,
        out_shape=jax.ShapeDtypeStruct((m_total, n_per), jnp.float32),
        in_specs=[
            pl.BlockSpec(memory_space=pltpu.VMEM),
            pl.BlockSpec(memory_space=pltpu.VMEM),
        ],
        out_specs=pl.BlockSpec(memory_space=pltpu.VMEM),
        scratch_shapes=[
            pltpu.VMEM((2, m_per, k), jnp.bfloat16),
            pltpu.SemaphoreType.DMA((2,)),
            pltpu.SemaphoreType.DMA((2,)),
            pltpu.VMEM((8, 128), jnp.float32),
            pltpu.VMEM((N_DEV, 8, 128), jnp.float32),
            pltpu.SemaphoreType.DMA((N_DEV - 1,)),
            pltpu.SemaphoreType.DMA((N_DEV,)),
        ],
        compiler_params=pltpu.CompilerParams(
            collective_id=0,
            vmem_limit_bytes=100 * 1024 * 1024,
        ),
    )(x, w_mat)


# device time: 171108 ns/iter; 1.7879x vs baseline; 1.7879x over previous
import jax
import jax.numpy as jnp
from jax import lax
from jax.experimental import pallas as pl
from jax.experimental.pallas import tpu as pltpu

N_DEV = 4


def kernel(x, w_mat):
    m_per, k = x.shape
    _, n_per = w_mat.shape
    m_total = N_DEV * m_per
    half = m_per // 2

    def body(x_ref, w_ref, out_ref,
             cw_comm, cw_send_sems, cw_recv_sems,
             ccw_comm, ccw_send_sems, ccw_recv_sems,
             amax_src, amax_dst, amax_send_sems, amax_recv_sems):
        my = lax.axis_index("i")
        left = lax.rem(my + N_DEV - 1, N_DEV)
        right = lax.rem(my + 1, N_DEV)

        barrier_sem = pltpu.get_barrier_semaphore()
        for nbr in (left, right):
            pl.semaphore_signal(barrier_sem, inc=1, device_id=(nbr,),
                                device_id_type=pl.DeviceIdType.MESH)
        pl.semaphore_wait(barrier_sem, 2)

        w_bf = w_ref[...].astype(jnp.bfloat16)
        cw_comm[0] = x_ref[:half, :].astype(jnp.bfloat16)
        ccw_comm[0] = x_ref[half:, :].astype(jnp.bfloat16)

        def block_out(xb):
            y = lax.dot_general(xb, w_bf, (((1,), (0,)), ((), ())),
                                preferred_element_type=jnp.float32)
            return jnp.maximum(y, 0.0)

        amax = jnp.float32(0.0)
        for h in range(N_DEV):
            s, r = h % 2, (h + 1) % 2
            rdmas = []
            if h < N_DEV - 1:
                cw = pltpu.make_async_remote_copy(
                    src_ref=cw_comm.at[s], dst_ref=cw_comm.at[r],
                    send_sem=cw_send_sems.at[s], recv_sem=cw_recv_sems.at[r],
                    device_id=(right,), device_id_type=pl.DeviceIdType.MESH,
                )
                ccw = pltpu.make_async_remote_copy(
                    src_ref=ccw_comm.at[s], dst_ref=ccw_comm.at[r],
                    send_sem=ccw_send_sems.at[s], recv_sem=ccw_recv_sems.at[r],
                    device_id=(left,), device_id_type=pl.DeviceIdType.MESH,
                )
                cw.start()
                ccw.start()
                rdmas = [cw, ccw]
            o_top = lax.rem(my + N_DEV - h, N_DEV)
            yb = block_out(cw_comm[s])
            out_ref[pl.ds(o_top * m_per, half), :] = yb
            amax = jnp.maximum(amax, jnp.max(yb))
            o_bot = lax.rem(my + h, N_DEV)
            yb = block_out(ccw_comm[s])
            out_ref[pl.ds(o_bot * m_per + half, half), :] = yb
            amax = jnp.maximum(amax, jnp.max(yb))
            for rd in rdmas:
                rd.wait()

        amax_src[...] = jnp.full((8, 128), amax, jnp.float32)
        sends = []
        for off in range(1, N_DEV):
            dst = lax.rem(my + off, N_DEV)
            snd = pltpu.make_async_remote_copy(
                src_ref=amax_src,
                dst_ref=amax_dst.at[my],
                send_sem=amax_send_sems.at[off - 1],
                recv_sem=amax_recv_sems.at[my],
                device_id=(dst,),
                device_id_type=pl.DeviceIdType.MESH,
            )
            snd.start()
            sends.append(snd)
        gmax = amax
        for off in range(1, N_DEV):
            src = lax.rem(my + off, N_DEV)
            rcv = pltpu.make_async_remote_copy(
                src_ref=amax_src,
                dst_ref=amax_dst.at[src],
                send_sem=amax_send_sems.at[off - 1],
                recv_sem=amax_recv_sems.at[src],
                device_id=(src,),
                device_id_type=pl.DeviceIdType.MESH,
            )
            rcv.wait_recv()
            gmax = jnp.maximum(gmax, jnp.max(amax_dst[src]))
        for snd in sends:
            snd.wait_send()

        scale = gmax / 127.0
        q = jnp.clip(jnp.round(out_ref[...] / scale), -127.0, 127.0)
        out_ref[...] = q * scale

    return pl.pallas_call(
        body,
        out_shape=jax.ShapeDtypeStruct((m_total, n_per), jnp.float32),
        in_specs=[
            pl.BlockSpec(memory_space=pltpu.VMEM),
            pl.BlockSpec(memory_space=pltpu.VMEM),
        ],
        out_specs=pl.BlockSpec(memory_space=pltpu.VMEM),
        scratch_shapes=[
            pltpu.VMEM((2, half, k), jnp.bfloat16),
            pltpu.SemaphoreType.DMA((2,)),
            pltpu.SemaphoreType.DMA((2,)),
            pltpu.VMEM((2, half, k), jnp.bfloat16),
            pltpu.SemaphoreType.DMA((2,)),
            pltpu.SemaphoreType.DMA((2,)),
            pltpu.VMEM((8, 128), jnp.float32),
            pltpu.VMEM((N_DEV, 8, 128), jnp.float32),
            pltpu.SemaphoreType.DMA((N_DEV - 1,)),
            pltpu.SemaphoreType.DMA((N_DEV,)),
        ],
        compiler_params=pltpu.CompilerParams(
            collective_id=0,
            vmem_limit_bytes=100 * 1024 * 1024,
        ),
    )(x, w_mat)


# device time: 169173 ns/iter; 1.8083x vs baseline; 1.0114x over previous
import jax
import jax.numpy as jnp
from jax import lax
from jax.experimental import pallas as pl
from jax.experimental.pallas import tpu as pltpu

N_DEV = 4


def kernel(x, w_mat):
    m_per, k = x.shape
    _, n_per = w_mat.shape
    m_total = N_DEV * m_per
    half = m_per // 2
    quarter = half // 2

    def body(x_ref, w_ref, out_ref,
             cw_comm, cw_send_sems, cw_recv_sems,
             ccw_comm, ccw_send_sems, ccw_recv_sems,
             amax_src, amax_dst, amax_send_sems, amax_recv_sems):
        my = lax.axis_index("i")
        left = lax.rem(my + N_DEV - 1, N_DEV)
        right = lax.rem(my + 1, N_DEV)

        barrier_sem = pltpu.get_barrier_semaphore()
        for nbr in (left, right):
            pl.semaphore_signal(barrier_sem, inc=1, device_id=(nbr,),
                                device_id_type=pl.DeviceIdType.MESH)
        pl.semaphore_wait(barrier_sem, 2)

        cw_comm[0] = x_ref[:half, :].astype(jnp.bfloat16)
        ccw_comm[0] = x_ref[half:, :].astype(jnp.bfloat16)

        def ring_pair(s, r, rows=None):
            if rows is None:
                sl = slice(None)
                cw_sems = (cw_send_sems.at[s], cw_recv_sems.at[r])
                ccw_sems = (ccw_send_sems.at[s], ccw_recv_sems.at[r])
            else:
                qi, sl = rows
                cw_sems = (cw_send_sems.at[2 + qi], cw_recv_sems.at[2 + qi])
                ccw_sems = (ccw_send_sems.at[2 + qi], ccw_recv_sems.at[2 + qi])
            cw = pltpu.make_async_remote_copy(
                src_ref=cw_comm.at[s, sl], dst_ref=cw_comm.at[r, sl],
                send_sem=cw_sems[0], recv_sem=cw_sems[1],
                device_id=(right,), device_id_type=pl.DeviceIdType.MESH,
            )
            ccw = pltpu.make_async_remote_copy(
                src_ref=ccw_comm.at[s, sl], dst_ref=ccw_comm.at[r, sl],
                send_sem=ccw_sems[0], recv_sem=ccw_sems[1],
                device_id=(left,), device_id_type=pl.DeviceIdType.MESH,
            )
            return cw, ccw

        def block_out(xb):
            y = lax.dot_general(xb, w_bf, (((1,), (0,)), ((), ())),
                                preferred_element_type=jnp.float32)
            return jnp.maximum(y, 0.0)

        hop0 = ring_pair(0, 1)
        for rd in hop0:
            rd.start()
        w_bf = w_ref[...].astype(jnp.bfloat16)

        def compute_step(h, s, amax, rows=slice(None), row_off=0, nrows=half):
            o_top = lax.rem(my + N_DEV - h, N_DEV)
            yb = block_out(cw_comm[s, rows])
            out_ref[pl.ds(o_top * m_per + row_off, nrows), :] = yb
            amax = jnp.maximum(amax, jnp.max(yb))
            o_bot = lax.rem(my + h, N_DEV)
            yb = block_out(ccw_comm[s, rows])
            out_ref[pl.ds(o_bot * m_per + half + row_off, nrows), :] = yb
            return jnp.maximum(amax, jnp.max(yb))

        amax = compute_step(0, 0, jnp.float32(0.0))
        for rd in hop0:
            rd.wait()

        hop1 = ring_pair(1, 0)
        for rd in hop1:
            rd.start()
        amax = compute_step(1, 1, amax)
        for rd in hop1:
            rd.wait()

        q0 = ring_pair(0, 1, rows=(0, pl.ds(0, quarter)))
        q1 = ring_pair(0, 1, rows=(1, pl.ds(quarter, quarter)))
        for rd in (*q0, *q1):
            rd.start()
        amax = compute_step(2, 0, amax)
        for rd in q0:
            rd.wait_recv()
        amax = compute_step(3, 1, amax, rows=pl.ds(0, quarter), nrows=quarter)
        for rd in q1:
            rd.wait_recv()
        amax = compute_step(3, 1, amax, rows=pl.ds(quarter, quarter),
                            row_off=quarter, nrows=quarter)
        for rd in (*q0, *q1):
            rd.wait_send()

        amax_src[...] = jnp.full((8, 128), amax, jnp.float32)
        sends = []
        for off in range(1, N_DEV):
            dst = lax.rem(my + off, N_DEV)
            snd = pltpu.make_async_remote_copy(
                src_ref=amax_src,
                dst_ref=amax_dst.at[my],
                send_sem=amax_send_sems.at[off - 1],
                recv_sem=amax_recv_sems.at[my],
                device_id=(dst,),
                device_id_type=pl.DeviceIdType.MESH,
            )
            snd.start()
            sends.append(snd)
        gmax = amax
        for off in range(1, N_DEV):
            src = lax.rem(my + off, N_DEV)
            rcv = pltpu.make_async_remote_copy(
                src_ref=amax_src,
                dst_ref=amax_dst.at[src],
                send_sem=amax_send_sems.at[off - 1],
                recv_sem=amax_recv_sems.at[src],
                device_id=(src,),
                device_id_type=pl.DeviceIdType.MESH,
            )
            rcv.wait_recv()
            gmax = jnp.maximum(gmax, jnp.max(amax_dst[src]))
        for snd in sends:
            snd.wait_send()

        scale = gmax / 127.0
        inv_scale = 127.0 / gmax
        q = jnp.clip(jnp.round(out_ref[...] * inv_scale), -127.0, 127.0)
        out_ref[...] = q * scale

    return pl.pallas_call(
        body,
        out_shape=jax.ShapeDtypeStruct((m_total, n_per), jnp.float32),
        in_specs=[
            pl.BlockSpec(memory_space=pltpu.VMEM),
            pl.BlockSpec(memory_space=pltpu.VMEM),
        ],
        out_specs=pl.BlockSpec(memory_space=pltpu.VMEM),
        scratch_shapes=[
            pltpu.VMEM((2, half, k), jnp.bfloat16),
            pltpu.SemaphoreType.DMA((4,)),
            pltpu.SemaphoreType.DMA((4,)),
            pltpu.VMEM((2, half, k), jnp.bfloat16),
            pltpu.SemaphoreType.DMA((4,)),
            pltpu.SemaphoreType.DMA((4,)),
            pltpu.VMEM((8, 128), jnp.float32),
            pltpu.VMEM((N_DEV, 8, 128), jnp.float32),
            pltpu.SemaphoreType.DMA((N_DEV - 1,)),
            pltpu.SemaphoreType.DMA((N_DEV,)),
        ],
        compiler_params=pltpu.CompilerParams(
            collective_id=0,
            vmem_limit_bytes=100 * 1024 * 1024,
        ),
    )(x, w_mat)


# device time: 165483 ns/iter; 1.8486x vs baseline; 1.0223x over previous
import jax
import jax.numpy as jnp
from jax import lax
from jax.experimental import pallas as pl
from jax.experimental.pallas import tpu as pltpu

N_DEV = 4
N_HOP = N_DEV - 1
N_Q = 2


def kernel(x, w_mat):
    m_per, k = x.shape
    _, n_per = w_mat.shape
    m_total = N_DEV * m_per
    half = m_per // 2
    quarter = half // N_Q

    def body(x_ref, w_ref, out_ref,
             cw_comm, cw_send_sems, cw_recv_sems,
             ccw_comm, ccw_send_sems, ccw_recv_sems,
             amax_src, amax_dst, amax_send_sems, amax_recv_sems):
        my = lax.axis_index("i")
        left = lax.rem(my + N_DEV - 1, N_DEV)
        right = lax.rem(my + 1, N_DEV)

        barrier_sem = pltpu.get_barrier_semaphore()
        for nbr in (left, right):
            pl.semaphore_signal(barrier_sem, inc=1, device_id=(nbr,),
                                device_id_type=pl.DeviceIdType.MESH)
        pl.semaphore_wait(barrier_sem, 2)

        def rdma(h, q, cw):
            comm = cw_comm if cw else ccw_comm
            ssem = cw_send_sems if cw else ccw_send_sems
            rsem = cw_recv_sems if cw else ccw_recv_sems
            rows = pl.ds(q * quarter, quarter)
            return pltpu.make_async_remote_copy(
                src_ref=comm.at[h % 3, rows],
                dst_ref=comm.at[(h + 1) % 3, rows],
                send_sem=ssem.at[h * N_Q + q],
                recv_sem=rsem.at[h * N_Q + q],
                device_id=(right if cw else left,),
                device_id_type=pl.DeviceIdType.MESH,
            )

        started = []
        for q in range(N_Q):
            rows = pl.ds(q * quarter, quarter)
            cw_comm[0, rows] = x_ref[rows, :].astype(jnp.bfloat16)
            d = rdma(0, q, cw=True)
            d.start()
            started.append(d)
            ccw_comm[0, rows] = x_ref[pl.ds(half + q * quarter, quarter),
                                      :].astype(jnp.bfloat16)
            d = rdma(0, q, cw=False)
            d.start()
            started.append(d)

        w_bf = w_ref[...].astype(jnp.bfloat16)

        def block_out(xb):
            y = lax.dot_general(xb, w_bf, (((1,), (0,)), ((), ())),
                                preferred_element_type=jnp.float32)
            return jnp.maximum(y, 0.0)

        def compute_step(h, amax, rows=slice(None), row_off=0, nrows=half):
            s = h % 3
            o_top = lax.rem(my + N_DEV - h, N_DEV)
            yb = block_out(cw_comm[s, rows])
            out_ref[pl.ds(o_top * m_per + row_off, nrows), :] = yb
            amax = jnp.maximum(amax, jnp.max(yb))
            o_bot = lax.rem(my + h, N_DEV)
            yb = block_out(ccw_comm[s, rows])
            out_ref[pl.ds(o_bot * m_per + half + row_off, nrows), :] = yb
            return jnp.maximum(amax, jnp.max(yb))

        amax = compute_step(0, jnp.float32(0.0))

        for h in range(1, N_HOP):
            for q in range(N_Q):
                for cw in (True, False):
                    rdma(h - 1, q, cw).wait_recv()
                    d = rdma(h, q, cw)
                    d.start()
                    started.append(d)
            amax = compute_step(h, amax)

        for q in range(N_Q):
            for cw in (True, False):
                rdma(N_HOP - 1, q, cw).wait_recv()
            amax = compute_step(N_HOP, amax,
                                rows=pl.ds(q * quarter, quarter),
                                row_off=q * quarter, nrows=quarter)

        amax_src[...] = jnp.full((8, 128), amax, jnp.float32)
        sends = []
        for off in range(1, N_DEV):
            dst = lax.rem(my + off, N_DEV)
            snd = pltpu.make_async_remote_copy(
                src_ref=amax_src,
                dst_ref=amax_dst.at[my],
                send_sem=amax_send_sems.at[off - 1],
                recv_sem=amax_recv_sems.at[my],
                device_id=(dst,),
                device_id_type=pl.DeviceIdType.MESH,
            )
            snd.start()
            sends.append(snd)
        gmax = amax
        for off in range(1, N_DEV):
            src = lax.rem(my + off, N_DEV)
            rcv = pltpu.make_async_remote_copy(
                src_ref=amax_src,
                dst_ref=amax_dst.at[src],
                send_sem=amax_send_sems.at[off - 1],
                recv_sem=amax_recv_sems.at[src],
                device_id=(src,),
                device_id_type=pl.DeviceIdType.MESH,
            )
            rcv.wait_recv()
            gmax = jnp.maximum(gmax, jnp.max(amax_dst[src]))
        for d in started + sends:
            d.wait_send()

        scale = gmax / 127.0
        inv_scale = 127.0 / gmax
        q = jnp.clip(jnp.round(out_ref[...] * inv_scale), -127.0, 127.0)
        out_ref[...] = q * scale

    return pl.pallas_call(
        body,
        out_shape=jax.ShapeDtypeStruct((m_total, n_per), jnp.float32),
        in_specs=[
            pl.BlockSpec(memory_space=pltpu.VMEM),
            pl.BlockSpec(memory_space=pltpu.VMEM),
        ],
        out_specs=pl.BlockSpec(memory_space=pltpu.VMEM),
        scratch_shapes=[
            pltpu.VMEM((3, half, k), jnp.bfloat16),
            pltpu.SemaphoreType.DMA((N_HOP * N_Q,)),
            pltpu.SemaphoreType.DMA((N_HOP * N_Q,)),
            pltpu.VMEM((3, half, k), jnp.bfloat16),
            pltpu.SemaphoreType.DMA((N_HOP * N_Q,)),
            pltpu.SemaphoreType.DMA((N_HOP * N_Q,)),
            pltpu.VMEM((8, 128), jnp.float32),
            pltpu.VMEM((N_DEV, 8, 128), jnp.float32),
            pltpu.SemaphoreType.DMA((N_DEV - 1,)),
            pltpu.SemaphoreType.DMA((N_DEV,)),
        ],
        compiler_params=pltpu.CompilerParams(
            collective_id=0,
            vmem_limit_bytes=100 * 1024 * 1024,
        ),
    )(x, w_mat)


# device time: 116638 ns/iter; 2.6228x vs baseline; 1.4188x over previous
import jax
import jax.numpy as jnp
from jax import lax
from jax.experimental import pallas as pl
from jax.experimental.pallas import tpu as pltpu

N_DEV = 4
N_HOP = N_DEV - 1


def kernel(x, w_mat):
    m_per, k = x.shape
    _, n_per = w_mat.shape
    m_total = N_DEV * m_per
    n_half = n_per // 2

    def body(x_ref, w_ref, out_ref,
             x_bf, y_buf, q_send, q_recv,
             cw_comm, cw_send_sems, cw_recv_sems,
             ccw_comm, ccw_send_sems, ccw_recv_sems,
             tile_send_sems, tile_recv_sems,
             amax_src, amax_dst, amax_send_sems, amax_recv_sems):
        my = lax.axis_index("i")
        left = lax.rem(my + N_DEV - 1, N_DEV)
        right = lax.rem(my + 1, N_DEV)

        barrier_sem = pltpu.get_barrier_semaphore()
        for nbr in (left, right):
            pl.semaphore_signal(barrier_sem, inc=1, device_id=(nbr,),
                                device_id_type=pl.DeviceIdType.MESH)
        pl.semaphore_wait(barrier_sem, 2)

        cw_comm[0] = w_ref[:, :n_half].astype(jnp.bfloat16)
        ccw_comm[0] = w_ref[:, n_half:].astype(jnp.bfloat16)

        def ring_pair(s, r):
            cw = pltpu.make_async_remote_copy(
                src_ref=cw_comm.at[s], dst_ref=cw_comm.at[r],
                send_sem=cw_send_sems.at[s], recv_sem=cw_recv_sems.at[r],
                device_id=(right,), device_id_type=pl.DeviceIdType.MESH,
            )
            ccw = pltpu.make_async_remote_copy(
                src_ref=ccw_comm.at[s], dst_ref=ccw_comm.at[r],
                send_sem=ccw_send_sems.at[s], recv_sem=ccw_recv_sems.at[r],
                device_id=(left,), device_id_type=pl.DeviceIdType.MESH,
            )
            return cw, ccw

        hop0 = ring_pair(0, 1)
        for d in hop0:
            d.start()
        x_bf[...] = x_ref[...].astype(jnp.bfloat16)

        def half_gemm(w_half):
            y = lax.dot_general(x_bf[...], w_half, (((1,), (0,)), ((), ())),
                                preferred_element_type=jnp.float32)
            return jnp.maximum(y, 0.0)

        def compute_step(h, amax):
            s = h % 2
            o_cw = lax.rem(my + N_DEV - h, N_DEV)
            yb = half_gemm(cw_comm[s])
            y_buf[o_cw, :, :n_half] = yb
            amax = jnp.maximum(amax, jnp.max(yb))
            o_ccw = lax.rem(my + h, N_DEV)
            yb = half_gemm(ccw_comm[s])
            y_buf[o_ccw, :, n_half:] = yb
            return jnp.maximum(amax, jnp.max(yb))

        amax = compute_step(0, jnp.float32(0.0))
        for d in hop0:
            d.wait()
        hop1 = ring_pair(1, 0)
        for d in hop1:
            d.start()
        amax = compute_step(1, amax)
        for d in hop1:
            d.wait()
        hop2 = ring_pair(0, 1)
        for d in hop2:
            d.start()
        amax = compute_step(2, amax)
        for d in hop2:
            d.wait()
        amax = compute_step(3, amax)

        amax_src[...] = jnp.full((8, 128), amax, jnp.float32)
        sends = []
        for off in range(1, N_DEV):
            dst = lax.rem(my + off, N_DEV)
            snd = pltpu.make_async_remote_copy(
                src_ref=amax_src,
                dst_ref=amax_dst.at[my],
                send_sem=amax_send_sems.at[off - 1],
                recv_sem=amax_recv_sems.at[my],
                device_id=(dst,),
                device_id_type=pl.DeviceIdType.MESH,
            )
            snd.start()
            sends.append(snd)
        gmax = amax
        for off in range(1, N_DEV):
            src = lax.rem(my + off, N_DEV)
            rcv = pltpu.make_async_remote_copy(
                src_ref=amax_src,
                dst_ref=amax_dst.at[src],
                send_sem=amax_send_sems.at[off - 1],
                recv_sem=amax_recv_sems.at[src],
                device_id=(src,),
                device_id_type=pl.DeviceIdType.MESH,
            )
            rcv.wait_recv()
            gmax = jnp.maximum(gmax, jnp.max(amax_dst[src]))

        scale = gmax / 127.0
        inv_scale = 127.0 / gmax

        def quantize(y):
            return jnp.clip(jnp.round(y.astype(jnp.float32) * inv_scale),
                            -127.0, 127.0).astype(jnp.int8)

        for off in range(1, N_DEV):
            dst = lax.rem(my + off, N_DEV)
            q_send[off - 1] = quantize(y_buf[dst])
            snd = pltpu.make_async_remote_copy(
                src_ref=q_send.at[off - 1],
                dst_ref=q_recv.at[my],
                send_sem=tile_send_sems.at[off - 1],
                recv_sem=tile_recv_sems.at[my],
                device_id=(dst,),
                device_id_type=pl.DeviceIdType.MESH,
            )
            snd.start()
            sends.append(snd)

        out_ref[pl.ds(my * m_per, m_per), :] = (
            quantize(y_buf[my]).astype(jnp.float32) * scale)

        for off in range(1, N_DEV):
            src = lax.rem(my + off, N_DEV)
            rcv = pltpu.make_async_remote_copy(
                src_ref=q_send.at[off - 1],
                dst_ref=q_recv.at[src],
                send_sem=tile_send_sems.at[off - 1],
                recv_sem=tile_recv_sems.at[src],
                device_id=(src,),
                device_id_type=pl.DeviceIdType.MESH,
            )
            rcv.wait_recv()
            out_ref[pl.ds(src * m_per, m_per), :] = (
                q_recv[src].astype(jnp.float32) * scale)

        for d in sends:
            d.wait_send()

    return pl.pallas_call(
        body,
        out_shape=jax.ShapeDtypeStruct((m_total, n_per), jnp.float32),
        in_specs=[
            pl.BlockSpec(memory_space=pltpu.VMEM),
            pl.BlockSpec(memory_space=pltpu.VMEM),
        ],
        out_specs=pl.BlockSpec(memory_space=pltpu.VMEM),
        scratch_shapes=[
            pltpu.VMEM((m_per, k), jnp.bfloat16),
            pltpu.VMEM((N_DEV, m_per, n_per), jnp.float32),
            pltpu.VMEM((N_DEV - 1, m_per, n_per), jnp.int8),
            pltpu.VMEM((N_DEV, m_per, n_per), jnp.int8),
            pltpu.VMEM((2, k, n_half), jnp.bfloat16),
            pltpu.SemaphoreType.DMA((2,)),
            pltpu.SemaphoreType.DMA((2,)),
            pltpu.VMEM((2, k, n_half), jnp.bfloat16),
            pltpu.SemaphoreType.DMA((2,)),
            pltpu.SemaphoreType.DMA((2,)),
            pltpu.SemaphoreType.DMA((N_DEV - 1,)),
            pltpu.SemaphoreType.DMA((N_DEV,)),
            pltpu.VMEM((8, 128), jnp.float32),
            pltpu.VMEM((N_DEV, 8, 128), jnp.float32),
            pltpu.SemaphoreType.DMA((N_DEV - 1,)),
            pltpu.SemaphoreType.DMA((N_DEV,)),
        ],
        compiler_params=pltpu.CompilerParams(
            collective_id=0,
            vmem_limit_bytes=100 * 1024 * 1024,
        ),
    )(x, w_mat)
